# baseline (speedup 1.0000x reference)
import functools

import jax
import jax.numpy as jnp
from jax import lax
from jax.experimental import pallas as pl
from jax.experimental.pallas import tpu as pltpu
from jax.experimental.pallas import tpu_sc as plsc

N_FZ, N_KEZ, N_ALPHA, N_TINT = 64, 8, 256, 128
NORB, NTIMES = 512, 256
NC, NS, L = 2, 16, 16
NW = NC * NS
T_TC = 96
T_SC = NTIMES - T_TC
TPW = T_SC // NW
JG = N_TINT // L
QG = NORB // L


def _prep_body(pf_ref, alpha_ref, dmag_ref, fzb_ref,
               s_ref, d_ref, m_ref, fzk_ref):
    la0 = pf_ref[0]
    inv_la = 1.0 / (pf_ref[1] - pf_ref[0])
    lf0 = pf_ref[2]
    inv_lf = 1.0 / (pf_ref[3] - pf_ref[2])
    amin = pf_ref[4]
    amax = pf_ref[5]

    a = alpha_ref[...]
    a_ind = (jnp.log10(a) - la0) * inv_la
    a0 = jnp.clip(a_ind.astype(jnp.int32), 0, N_ALPHA - 1)
    d_ref[...] = jnp.swapaxes(a_ind - a0.astype(jnp.float32),
                              0, 1).reshape(NTIMES, 1, NORB)
    s_ref[...] = jnp.swapaxes(jnp.minimum(a0, N_ALPHA - 2),
                              0, 1).reshape(NTIMES, 1, NORB)
    geom = (a >= amin) & (a <= amax)
    m_ref[...] = jnp.swapaxes(
        jnp.where(geom, dmag_ref[...], jnp.float32(jnp.inf)),
        0, 1).reshape(NTIMES, 1, NORB)

    fz_ind = (jnp.log10(fzb_ref[...]) - lf0) * inv_lf
    fzk_ref[...] = jnp.clip(jnp.floor(fz_ind).astype(jnp.int32) + 1,
                            0, N_FZ - 2).reshape(NTIMES, 1, L)


_prep = pl.pallas_call(
    _prep_body,
    in_specs=[
        pl.BlockSpec(memory_space=pltpu.SMEM),
        pl.BlockSpec(memory_space=pltpu.VMEM),
        pl.BlockSpec(memory_space=pltpu.VMEM),
        pl.BlockSpec(memory_space=pltpu.VMEM),
    ],
    out_specs=[
        pl.BlockSpec(memory_space=pltpu.VMEM),
        pl.BlockSpec(memory_space=pltpu.VMEM),
        pl.BlockSpec(memory_space=pltpu.VMEM),
        pl.BlockSpec(memory_space=pltpu.VMEM),
    ],
    out_shape=[
        jax.ShapeDtypeStruct((NTIMES, 1, NORB), jnp.int32),
        jax.ShapeDtypeStruct((NTIMES, 1, NORB), jnp.float32),
        jax.ShapeDtypeStruct((NTIMES, 1, NORB), jnp.float32),
        jax.ShapeDtypeStruct((NTIMES, 1, L), jnp.int32),
    ],
)


def _tc_body(fzk_ref, kz_ref, s_ref, d_ref, m_ref, grid_ref, out_ref):
    del fzk_ref, kz_ref
    s2 = s_ref[0]
    d2 = d_ref[0]
    m2 = m_ref[0]
    slab = grid_ref[0, 0]
    ai = lax.broadcasted_iota(jnp.int32, (N_ALPHA, NORB), 0)
    w = (jnp.where(ai == s2, 1.0 - d2, 0.0)
         + jnp.where(ai == s2 + 1, d2, 0.0))
    dim_t = lax.dot_general(slab, w, (((0,), (0,)), ((), ())),
                            preferred_element_type=jnp.float32)
    hits = (m2 < dim_t).astype(jnp.float32)
    out_ref[...] = (jnp.sum(hits, axis=1)
                    * (1.0 / NORB)).reshape(1, 1, N_TINT)


_tc_main = pl.pallas_call(
    _tc_body,
    grid_spec=pltpu.PrefetchScalarGridSpec(
        num_scalar_prefetch=2,
        grid=(T_TC,),
        in_specs=[
            pl.BlockSpec((1, 1, NORB), lambda t, fzk, kz: (T_SC + t, 0, 0)),
            pl.BlockSpec((1, 1, NORB), lambda t, fzk, kz: (T_SC + t, 0, 0)),
            pl.BlockSpec((1, 1, NORB), lambda t, fzk, kz: (T_SC + t, 0, 0)),
            pl.BlockSpec((1, 1, N_ALPHA, N_TINT),
                         lambda t, fzk, kz: (fzk[T_SC + t, 0, 0],
                                             kz[0], 0, 0)),
        ],
        out_specs=pl.BlockSpec((1, 1, N_TINT),
                               lambda t, fzk, kz: (t, 0, 0)),
    ),
    out_shape=jax.ShapeDtypeStruct((T_TC, 1, N_TINT), jnp.float32),
)


@functools.partial(
    pl.kernel,
    mesh=plsc.VectorSubcoreMesh(core_axis_name="c", subcore_axis_name="s"),
    out_type=jax.ShapeDtypeStruct((T_SC, 1, N_TINT), jnp.float32),
    scratch_types=[
        pltpu.VMEM((TPW, 1, NORB), jnp.int32),
        pltpu.VMEM((TPW, 1, NORB), jnp.float32),
        pltpu.VMEM((TPW, 1, NORB), jnp.float32),
        pltpu.VMEM((TPW, 1, L), jnp.int32),
        pltpu.VMEM((2 * N_ALPHA * N_TINT,), jnp.float32),
        pltpu.VMEM((TPW, 1, N_TINT), jnp.float32),
        pltpu.SemaphoreType.DMA,
        pltpu.SemaphoreType.DMA,
    ],
)
def _sc_main(s_hbm, d_hbm, m_hbm, fzk_hbm, table_hbm, out_hbm,
             s_v, d_v, m_v, fz_v, slab_v, outbuf_v, sem0, sem1):
    wid = lax.axis_index("s") * NC + lax.axis_index("c")
    t0 = wid * TPW
    pltpu.sync_copy(fzk_hbm.at[pl.ds(t0, TPW)], fz_v)
    sems = (sem0, sem1)
    SLAB = N_ALPHA * N_TINT

    def start_slab(it, b):
        fzk = fz_v[it, 0, pl.ds(0, L)][0]
        return pltpu.async_copy(table_hbm.at[fzk],
                                slab_v.at[pl.ds(b * SLAB, SLAB)], sems[b])

    cds = [start_slab(0, 0), None]
    pltpu.sync_copy(s_hbm.at[pl.ds(t0, TPW)], s_v)
    pltpu.sync_copy(d_hbm.at[pl.ds(t0, TPW)], d_v)
    pltpu.sync_copy(m_hbm.at[pl.ds(t0, TPW)], m_v)

    for it in range(TPW):
        b = it % 2
        cds[b].wait()
        if it + 1 < TPW:
            cds[(it + 1) % 2] = start_slab(it + 1, (it + 1) % 2)

        def qbody(g, accs, it=it, b=b):
            qb = g * L
            sv = s_v[it, 0, pl.ds(qb, L)]
            dv = d_v[it, 0, pl.ds(qb, L)]
            mv = m_v[it, 0, pl.ds(qb, L)]
            accs = list(accs)
            for l in range(L):
                off = sv[l] * N_TINT + b * SLAB
                dd = dv[l]
                mm = mv[l]
                for j in range(JG):
                    v0 = slab_v[pl.ds(off + j * L, L)]
                    v1 = slab_v[pl.ds(off + N_TINT + j * L, L)]
                    dim = v0 + dd * (v1 - v0)
                    hit = jnp.where(mm < dim, jnp.float32(1.0),
                                    jnp.float32(0.0))
                    accs[j] = accs[j] + hit
            return tuple(accs)

        zeros = tuple(jnp.zeros((L,), jnp.float32) for _ in range(JG))
        accs = lax.fori_loop(0, QG, qbody, zeros)
        scale = jnp.float32(1.0 / NORB)
        for j in range(JG):
            outbuf_v[it, 0, pl.ds(j * L, L)] = accs[j] * scale

    pltpu.sync_copy(outbuf_v, out_hbm.at[pl.ds(t0, TPW)])


def kernel(alpha, dMag, fZ_vals, kEZ_val, fZs, kEZs, alphas, int_times, grid):
    kz = jnp.clip(jnp.sum((kEZs <= kEZ_val).astype(jnp.int32)) - 1,
                  0, N_KEZ - 1)
    lg4 = jnp.log10(jnp.concatenate([alphas[:2], fZs[:2]]))
    pf = jnp.concatenate([lg4, alphas[:1], alphas[-1:]]).astype(jnp.float32)

    fzb = jnp.broadcast_to(fZ_vals.astype(jnp.float32)[:, None], (NTIMES, L))

    s, d, m, fzk = _prep(pf, alpha, dMag, fzb)
    table = lax.dynamic_index_in_dim(grid, kz, axis=1, keepdims=False)
    table = table.reshape(N_FZ, N_ALPHA * N_TINT)
    out_sc = _sc_main(s, d, m, fzk, table).reshape(T_SC, N_TINT)
    out_tc = _tc_main(fzk, kz.reshape((1,)), s, d, m, grid)
    return jnp.concatenate([out_sc, out_tc.reshape(T_TC, N_TINT)], axis=0)


# speedup vs baseline: 1.2410x; 1.2410x over previous
import functools

import jax
import jax.numpy as jnp
from jax import lax
from jax.experimental import pallas as pl
from jax.experimental.pallas import tpu as pltpu
from jax.experimental.pallas import tpu_sc as plsc

N_FZ, N_KEZ, N_ALPHA, N_TINT = 64, 8, 256, 128
NORB, NTIMES = 512, 256
NC, NS, L = 2, 16, 16
NW = NC * NS
T_TC = 64
T_SC = NTIMES - T_TC
TPW = T_SC // NW
JG = N_TINT // L
QG = NORB // L


def _prep_body(pf_ref, alpha_ref, dmag_ref, fzb_ref,
               s_ref, d_ref, m_ref, fzk_ref):
    la0 = pf_ref[0]
    inv_la = 1.0 / (pf_ref[1] - pf_ref[0])
    lf0 = pf_ref[2]
    inv_lf = 1.0 / (pf_ref[3] - pf_ref[2])
    amin = pf_ref[4]
    amax = pf_ref[5]

    a = alpha_ref[...]
    a_ind = (jnp.log10(a) - la0) * inv_la
    a0 = jnp.clip(a_ind.astype(jnp.int32), 0, N_ALPHA - 1)
    d_ref[...] = jnp.swapaxes(a_ind - a0.astype(jnp.float32),
                              0, 1).reshape(NTIMES, 1, NORB)
    s_ref[...] = jnp.swapaxes(jnp.minimum(a0, N_ALPHA - 2),
                              0, 1).reshape(NTIMES, 1, NORB)
    geom = (a >= amin) & (a <= amax)
    m_ref[...] = jnp.swapaxes(
        jnp.where(geom, dmag_ref[...], jnp.float32(jnp.inf)),
        0, 1).reshape(NTIMES, 1, NORB)

    fz_ind = (jnp.log10(fzb_ref[...]) - lf0) * inv_lf
    fzk_ref[...] = jnp.clip(jnp.floor(fz_ind).astype(jnp.int32) + 1,
                            0, N_FZ - 2).reshape(NTIMES, 1, L)


_prep = pl.pallas_call(
    _prep_body,
    in_specs=[
        pl.BlockSpec(memory_space=pltpu.SMEM),
        pl.BlockSpec(memory_space=pltpu.VMEM),
        pl.BlockSpec(memory_space=pltpu.VMEM),
        pl.BlockSpec(memory_space=pltpu.VMEM),
    ],
    out_specs=[
        pl.BlockSpec(memory_space=pltpu.VMEM),
        pl.BlockSpec(memory_space=pltpu.VMEM),
        pl.BlockSpec(memory_space=pltpu.VMEM),
        pl.BlockSpec(memory_space=pltpu.VMEM),
    ],
    out_shape=[
        jax.ShapeDtypeStruct((NTIMES, 1, NORB), jnp.int32),
        jax.ShapeDtypeStruct((NTIMES, 1, NORB), jnp.float32),
        jax.ShapeDtypeStruct((NTIMES, 1, NORB), jnp.float32),
        jax.ShapeDtypeStruct((NTIMES, 1, L), jnp.int32),
    ],
)


def _tc_body(fzk_ref, kz_ref, pm_ref, s_ref, d_ref, m_ref, grid_ref, out_ref):
    del fzk_ref, kz_ref, pm_ref
    s2 = s_ref[0]
    d2 = d_ref[0]
    m2 = m_ref[0]
    slab = grid_ref[0, 0]
    ai = lax.broadcasted_iota(jnp.int32, (N_ALPHA, NORB), 0)
    w = (jnp.where(ai == s2, 1.0 - d2, 0.0)
         + jnp.where(ai == s2 + 1, d2, 0.0))
    dim_t = lax.dot_general(slab, w, (((0,), (0,)), ((), ())),
                            preferred_element_type=jnp.float32)
    hits = (m2 < dim_t).astype(jnp.float32)
    out_ref[...] = (jnp.sum(hits, axis=1)
                    * (1.0 / NORB)).reshape(1, 1, N_TINT)


_tc_main = pl.pallas_call(
    _tc_body,
    grid_spec=pltpu.PrefetchScalarGridSpec(
        num_scalar_prefetch=3,
        grid=(T_TC,),
        in_specs=[
            pl.BlockSpec((1, 1, NORB),
                         lambda t, fzk, kz, pm: (T_SC + pm[t], 0, 0)),
            pl.BlockSpec((1, 1, NORB),
                         lambda t, fzk, kz, pm: (T_SC + pm[t], 0, 0)),
            pl.BlockSpec((1, 1, NORB),
                         lambda t, fzk, kz, pm: (T_SC + pm[t], 0, 0)),
            pl.BlockSpec((1, 1, N_ALPHA, N_TINT),
                         lambda t, fzk, kz, pm: (fzk[T_SC + pm[t], 0, 0],
                                                 kz[0], 0, 0)),
        ],
        out_specs=pl.BlockSpec((1, 1, N_TINT),
                               lambda t, fzk, kz, pm: (pm[t], 0, 0)),
    ),
    out_shape=jax.ShapeDtypeStruct((T_TC, 1, N_TINT), jnp.float32),
)


@functools.partial(
    pl.kernel,
    mesh=plsc.VectorSubcoreMesh(core_axis_name="c", subcore_axis_name="s"),
    out_type=jax.ShapeDtypeStruct((T_SC, 1, N_TINT), jnp.float32),
    scratch_types=[
        pltpu.VMEM((TPW, 1, NORB), jnp.int32),
        pltpu.VMEM((TPW, 1, NORB), jnp.float32),
        pltpu.VMEM((TPW, 1, NORB), jnp.float32),
        pltpu.VMEM((TPW, 1, L), jnp.int32),
        pltpu.VMEM((2 * N_ALPHA * N_TINT,), jnp.float32),
        pltpu.VMEM((TPW, 1, N_TINT), jnp.float32),
        pltpu.SemaphoreType.DMA,
        pltpu.SemaphoreType.DMA,
    ],
)
def _sc_main(s_hbm, d_hbm, m_hbm, fzk_hbm, table_hbm, out_hbm,
             s_v, d_v, m_v, fz_v, slab_v, outbuf_v, sem0, sem1):
    wid = lax.axis_index("s") * NC + lax.axis_index("c")
    t0 = wid * TPW
    pltpu.sync_copy(fzk_hbm.at[pl.ds(t0, TPW)], fz_v)
    sems = (sem0, sem1)
    SLAB = N_ALPHA * N_TINT

    def start_slab(it, b):
        fzk = fz_v[it, 0, pl.ds(0, L)][0]
        return pltpu.async_copy(table_hbm.at[fzk],
                                slab_v.at[pl.ds(b * SLAB, SLAB)], sems[b])

    cds = [start_slab(0, 0), None]
    pltpu.sync_copy(s_hbm.at[pl.ds(t0, TPW)], s_v)
    pltpu.sync_copy(d_hbm.at[pl.ds(t0, TPW)], d_v)
    pltpu.sync_copy(m_hbm.at[pl.ds(t0, TPW)], m_v)

    for it in range(TPW):
        b = it % 2
        cds[b].wait()
        if it + 1 < TPW:
            cds[(it + 1) % 2] = start_slab(it + 1, (it + 1) % 2)

        def qbody(g, accs, it=it, b=b):
            qb = g * L
            sv = s_v[it, 0, pl.ds(qb, L)]
            dv = d_v[it, 0, pl.ds(qb, L)]
            mv = m_v[it, 0, pl.ds(qb, L)]
            accs = list(accs)
            for l in range(L):
                off = sv[l] * N_TINT + b * SLAB
                dd = dv[l]
                mm = mv[l]
                for j in range(JG):
                    v0 = slab_v[pl.ds(off + j * L, L)]
                    v1 = slab_v[pl.ds(off + N_TINT + j * L, L)]
                    dim = v0 + dd * (v1 - v0)
                    hit = jnp.where(mm < dim, jnp.float32(1.0),
                                    jnp.float32(0.0))
                    accs[j] = accs[j] + hit
            return tuple(accs)

        zeros = tuple(jnp.zeros((L,), jnp.float32) for _ in range(JG))
        accs = lax.fori_loop(0, QG, qbody, zeros)
        scale = jnp.float32(1.0 / NORB)
        for j in range(JG):
            outbuf_v[it, 0, pl.ds(j * L, L)] = accs[j] * scale

    pltpu.sync_copy(outbuf_v, out_hbm.at[pl.ds(t0, TPW)])


def kernel(alpha, dMag, fZ_vals, kEZ_val, fZs, kEZs, alphas, int_times, grid):
    kz = jnp.clip(jnp.sum((kEZs <= kEZ_val).astype(jnp.int32)) - 1,
                  0, N_KEZ - 1)
    lg4 = jnp.log10(jnp.concatenate([alphas[:2], fZs[:2]]))
    pf = jnp.concatenate([lg4, alphas[:1], alphas[-1:]]).astype(jnp.float32)

    fzb = jnp.broadcast_to(fZ_vals.astype(jnp.float32)[:, None], (NTIMES, L))

    s, d, m, fzk = _prep(pf, alpha, dMag, fzb)
    table = lax.dynamic_index_in_dim(grid, kz, axis=1, keepdims=False)
    table = table.reshape(N_FZ, N_ALPHA * N_TINT)
    out_sc = _sc_main(s, d, m, fzk, table).reshape(T_SC, N_TINT)
    perm = jnp.argsort(lax.slice_in_dim(fZ_vals, T_SC, NTIMES)
                       ).astype(jnp.int32)
    out_tc = _tc_main(fzk, kz.reshape((1,)), perm, s, d, m, grid)
    return jnp.concatenate([out_sc, out_tc.reshape(T_TC, N_TINT)], axis=0)
